# Initial kernel scaffold; baseline (speedup 1.0000x reference)
import functools

import jax
import jax.numpy as jnp
from jax import lax
from jax.experimental import pallas as pl
from jax.experimental.pallas import tpu as pltpu
from jax.experimental.pallas import tpu_sc as plsc

N = 10000
E = 320000
D1 = 16
D2P = 48
NC = 2
NS = 16
CH = 80
CPT = E // CH // (NC * NS)
BR = 1000
GRID = N // BR


def _sc_mesh():
    return plsc.VectorSubcoreMesh(
        core_axis_name="c", subcore_axis_name="s", num_cores=NC, num_subcores=NS
    )


def _sc_degree(dst2d, ones_ch, zeros_n):
    @functools.partial(
        pl.kernel,
        out_type=jax.ShapeDtypeStruct((NC, N), jnp.float32),
        mesh=_sc_mesh(),
        scratch_types=[
            pltpu.VMEM((CPT, CH), jnp.int32),
            pltpu.VMEM((CH,), jnp.float32),
            pltpu.VMEM_SHARED((N,), jnp.float32),
        ],
    )
    def body(dst_hbm, ones_hbm, zeros_hbm, out_hbm, idst, ones_v, acc):
        cid = lax.axis_index("c")
        sid = lax.axis_index("s")
        wid = cid * NS + sid
        pltpu.sync_copy(dst_hbm.at[pl.ds(wid * CPT, CPT)], idst)
        pltpu.sync_copy(ones_hbm, ones_v)
        @pl.when(sid < 5)
        def _():
            pltpu.sync_copy(
                zeros_hbm.at[pl.ds(sid * 2000, 2000)], acc.at[pl.ds(sid * 2000, 2000)]
            )
        plsc.subcore_barrier()

        def step(j, carry):
            pltpu.sync_copy(ones_v, acc.at[idst.at[j]], add=True)
            return carry

        lax.fori_loop(0, CPT, step, 0)
        plsc.subcore_barrier()

        @pl.when(sid < 5)
        def _():
            pltpu.sync_copy(
                acc.at[pl.ds(sid * 2000, 2000)],
                out_hbm.at[cid].at[pl.ds(sid * 2000, 2000)],
            )

    return body(dst2d, ones_ch, zeros_n)


def _sc_agg(y, src2d, dst2d, zeros_nd, d):
    rows_per_tile = N // NS

    @functools.partial(
        pl.kernel,
        out_type=jax.ShapeDtypeStruct((NC, N, d), jnp.float32),
        mesh=_sc_mesh(),
        scratch_types=[
            pltpu.VMEM((CPT, CH), jnp.int32),
            pltpu.VMEM((CPT, CH), jnp.int32),
            pltpu.VMEM((CH, d), jnp.float32),
            pltpu.VMEM_SHARED((N, d), jnp.float32),
            pltpu.SemaphoreType.DMA,
        ],
    )
    def body(y_hbm, src_hbm, dst_hbm, zeros_hbm, out_hbm, isrc, idst, buf, acc, sem):
        cid = lax.axis_index("c")
        sid = lax.axis_index("s")
        wid = cid * NS + sid
        pltpu.sync_copy(src_hbm.at[pl.ds(wid * CPT, CPT)], isrc)
        pltpu.sync_copy(dst_hbm.at[pl.ds(wid * CPT, CPT)], idst)
        pltpu.sync_copy(
            zeros_hbm.at[pl.ds(sid * rows_per_tile, rows_per_tile)],
            acc.at[pl.ds(sid * rows_per_tile, rows_per_tile)],
        )
        plsc.subcore_barrier()

        def step(j, carry):
            pltpu.async_copy(y_hbm.at[isrc.at[j]], buf, sem).wait()
            pltpu.sync_copy(buf, acc.at[idst.at[j]], add=True)
            return carry

        lax.fori_loop(0, CPT, step, 0)
        plsc.subcore_barrier()
        pltpu.sync_copy(
            acc.at[pl.ds(sid * rows_per_tile, rows_per_tile)],
            out_hbm.at[cid].at[pl.ds(sid * rows_per_tile, rows_per_tile)],
        )

    return body(y, src2d, dst2d, zeros_nd)


def _dinv_of(degp_ref):
    deg = degp_ref[0] + degp_ref[1] + 1.0
    return lax.rsqrt(deg)


def _tc_y1(x, W1, degp):
    def body(x_ref, w_ref, degp_ref, y1_ref):
        dinv = _dinv_of(degp_ref)
        xw = jnp.dot(x_ref[...], w_ref[...], preferred_element_type=jnp.float32)
        y1_ref[...] = xw * dinv[:, None]

    return pl.pallas_call(
        body,
        grid=(GRID,),
        in_specs=[
            pl.BlockSpec((BR, 128), lambda i: (i, 0)),
            pl.BlockSpec((128, D1), lambda i: (0, 0)),
            pl.BlockSpec((NC, BR), lambda i: (0, i)),
        ],
        out_specs=pl.BlockSpec((BR, D1), lambda i: (i, 0)),
        out_shape=jax.ShapeDtypeStruct((N, D1), jnp.float32),
    )(x, W1, degp)


def _tc_mid(aggp, y1, degp, W2, b1):
    def body(aggp_ref, y1_ref, degp_ref, w_ref, b1_ref, y2_ref):
        dinv = _dinv_of(degp_ref)
        agg = aggp_ref[0] + aggp_ref[1] + y1_ref[...]
        pre = agg * dinv[:, None] + b1_ref[...]
        h = jnp.where(pre > 0, pre, jnp.expm1(pre))
        hw = jnp.dot(h, w_ref[...], preferred_element_type=jnp.float32)
        y2 = hw * dinv[:, None]
        y2_ref[...] = jnp.concatenate(
            [y2, jnp.zeros((BR, D2P - 40), jnp.float32)], axis=1
        )

    return pl.pallas_call(
        body,
        grid=(GRID,),
        in_specs=[
            pl.BlockSpec((NC, BR, D1), lambda i: (0, i, 0)),
            pl.BlockSpec((BR, D1), lambda i: (i, 0)),
            pl.BlockSpec((NC, BR), lambda i: (0, i)),
            pl.BlockSpec((D1, 40), lambda i: (0, 0)),
            pl.BlockSpec((1, D1), lambda i: (0, 0)),
        ],
        out_specs=pl.BlockSpec((BR, D2P), lambda i: (i, 0)),
        out_shape=jax.ShapeDtypeStruct((N, D2P), jnp.float32),
    )(aggp, y1, degp, W2, b1)


def _tc_final(aggp2, y2, degp, b2p):
    def body(aggp_ref, y2_ref, degp_ref, b2_ref, out_ref):
        dinv = _dinv_of(degp_ref)
        agg = aggp_ref[0] + aggp_ref[1] + y2_ref[...]
        z = agg * dinv[:, None] + b2_ref[...]
        lane = lax.broadcasted_iota(jnp.int32, (BR, D2P), 1)
        zm = jnp.where(lane < 40, z, -1e30)
        m = jnp.max(zm, axis=1, keepdims=True)
        s = jnp.log(jnp.sum(jnp.exp(zm - m), axis=1, keepdims=True))
        out_ref[...] = z - m - s

    return pl.pallas_call(
        body,
        grid=(GRID,),
        in_specs=[
            pl.BlockSpec((NC, BR, D2P), lambda i: (0, i, 0)),
            pl.BlockSpec((BR, D2P), lambda i: (i, 0)),
            pl.BlockSpec((NC, BR), lambda i: (0, i)),
            pl.BlockSpec((1, D2P), lambda i: (0, 0)),
        ],
        out_specs=pl.BlockSpec((BR, D2P), lambda i: (i, 0)),
        out_shape=jax.ShapeDtypeStruct((N, D2P), jnp.float32),
    )(aggp2, y2, degp, b2p)


def kernel(node_feature, adj_mat, W1, b1, W2, b2):
    src2d = adj_mat[0].reshape(E // CH, CH)
    dst2d = adj_mat[1].reshape(E // CH, CH)
    ones_ch = jnp.ones((CH,), jnp.float32)
    zeros_n = jnp.zeros((N,), jnp.float32)
    zeros_n16 = jnp.zeros((N, D1), jnp.float32)
    zeros_n48 = jnp.zeros((N, D2P), jnp.float32)
    b1r = b1.reshape(1, D1)
    b2p = jnp.concatenate([b2, jnp.zeros((D2P - 40,), jnp.float32)]).reshape(1, D2P)

    degp = _sc_degree(dst2d, ones_ch, zeros_n)
    y1 = _tc_y1(node_feature, W1, degp)
    aggp1 = _sc_agg(y1, src2d, dst2d, zeros_n16, D1)
    y2 = _tc_mid(aggp1, y1, degp, W2, b1r)
    aggp2 = _sc_agg(y2, src2d, dst2d, zeros_n48, D2P)
    out48 = _tc_final(aggp2, y2, degp, b2p)
    return out48[:, :40]


# speedup vs baseline: 25.9853x; 25.9853x over previous
import functools

import jax
import jax.numpy as jnp
from jax import lax
from jax.experimental import pallas as pl
from jax.experimental.pallas import tpu as pltpu
from jax.experimental.pallas import tpu_sc as plsc

N = 10000
E = 320000
D1 = 16
D2P = 48
NC = 2
NS = 16
CH = 125
CPT = E // CH // (NC * NS)
BR = 1000
GRID = N // BR


def _sc_mesh():
    return plsc.VectorSubcoreMesh(
        core_axis_name="c", subcore_axis_name="s", num_cores=NC, num_subcores=NS
    )


_SC_PARAMS = pltpu.CompilerParams(use_tc_tiling_on_sc=False)


def _sc_degree(dst2d, ones_ch, zeros_n16):
    @functools.partial(
        pl.kernel,
        out_type=jax.ShapeDtypeStruct((NC, N, D1), jnp.float32),
        mesh=_sc_mesh(),
        compiler_params=_SC_PARAMS,
        scratch_types=[
            pltpu.VMEM((CPT, CH), jnp.int32),
            pltpu.VMEM((CH, D1), jnp.float32),
            pltpu.VMEM_SHARED((N, D1), jnp.float32),
        ],
    )
    def body(dst_hbm, ones_hbm, zeros_hbm, out_hbm, idst, ones_v, acc):
        cid = lax.axis_index("c")
        sid = lax.axis_index("s")
        wid = cid * NS + sid
        pltpu.sync_copy(dst_hbm.at[pl.ds(wid * CPT, CPT)], idst)
        pltpu.sync_copy(ones_hbm, ones_v)
        @pl.when(sid < 10)
        def _():
            pltpu.sync_copy(
                zeros_hbm.at[pl.ds(sid * 1000, 1000)], acc.at[pl.ds(sid * 1000, 1000)]
            )
        plsc.subcore_barrier()

        def step(j, carry):
            pltpu.sync_copy(ones_v, acc.at[idst.at[j]], add=True)
            return carry

        lax.fori_loop(0, CPT, step, 0)
        plsc.subcore_barrier()

        @pl.when(sid < 10)
        def _():
            pltpu.sync_copy(
                acc.at[pl.ds(sid * 1000, 1000)],
                out_hbm.at[cid].at[pl.ds(sid * 1000, 1000)],
            )

    return body(dst2d, ones_ch, zeros_n16)


def _sc_agg(y, src2d, dst2d, zeros_nd, d):
    rpt = 1000

    @functools.partial(
        pl.kernel,
        out_type=jax.ShapeDtypeStruct((NC, N, d), jnp.float32),
        mesh=_sc_mesh(),
        compiler_params=_SC_PARAMS,
        scratch_types=[
            pltpu.VMEM((CPT, CH), jnp.int32),
            pltpu.VMEM((CPT, CH), jnp.int32),
            pltpu.VMEM((CH, d), jnp.float32),
            pltpu.VMEM_SHARED((N, d), jnp.float32),
            pltpu.SemaphoreType.DMA,
        ],
    )
    def body(y_hbm, src_hbm, dst_hbm, zeros_hbm, out_hbm, isrc, idst, buf, acc, sem):
        cid = lax.axis_index("c")
        sid = lax.axis_index("s")
        wid = cid * NS + sid
        pltpu.sync_copy(src_hbm.at[pl.ds(wid * CPT, CPT)], isrc)
        pltpu.sync_copy(dst_hbm.at[pl.ds(wid * CPT, CPT)], idst)

        @pl.when(sid < 10)
        def _():
            pltpu.sync_copy(
                zeros_hbm.at[pl.ds(sid * rpt, rpt)], acc.at[pl.ds(sid * rpt, rpt)]
            )
        plsc.subcore_barrier()

        def step(j, carry):
            pltpu.async_copy(y_hbm.at[isrc.at[j]], buf, sem).wait()
            pltpu.sync_copy(buf, acc.at[idst.at[j]], add=True)
            return carry

        lax.fori_loop(0, CPT, step, 0)
        plsc.subcore_barrier()

        @pl.when(sid < 10)
        def _():
            pltpu.sync_copy(
                acc.at[pl.ds(sid * rpt, rpt)],
                out_hbm.at[cid].at[pl.ds(sid * rpt, rpt)],
            )

    return body(y, src2d, dst2d, zeros_nd)


def _dinv_of(degp_ref):
    deg = jnp.sum(degp_ref[...], axis=1) + 1.0
    return lax.rsqrt(deg)


def _tc_y1(x, W1, degp):
    def body(x_ref, w_ref, degp_ref, y1_ref):
        dinv = _dinv_of(degp_ref)
        xw = jnp.dot(x_ref[...], w_ref[...], preferred_element_type=jnp.float32)
        y1_ref[...] = xw * dinv[:, None]

    return pl.pallas_call(
        body,
        grid=(GRID,),
        in_specs=[
            pl.BlockSpec((BR, 128), lambda i: (i, 0)),
            pl.BlockSpec((128, D1), lambda i: (0, 0)),
            pl.BlockSpec((BR, NC), lambda i: (i, 0)),
        ],
        out_specs=pl.BlockSpec((BR, D1), lambda i: (i, 0)),
        out_shape=jax.ShapeDtypeStruct((N, D1), jnp.float32),
    )(x, W1, degp)


def _tc_mid(aggp, y1, degp, W2, b1):
    def body(aggp_ref, y1_ref, degp_ref, w_ref, b1_ref, y2_ref):
        dinv = _dinv_of(degp_ref)
        agg = aggp_ref[0] + aggp_ref[1] + y1_ref[...]
        pre = agg * dinv[:, None] + b1_ref[...]
        h = jnp.where(pre > 0, pre, jnp.exp(jnp.minimum(pre, 0.0)) - 1.0)
        hw = jnp.dot(h, w_ref[...], preferred_element_type=jnp.float32)
        y2 = hw * dinv[:, None]
        y2_ref[...] = jnp.concatenate(
            [y2, jnp.zeros((BR, D2P - 40), jnp.float32)], axis=1
        )

    return pl.pallas_call(
        body,
        grid=(GRID,),
        in_specs=[
            pl.BlockSpec((NC, BR, D1), lambda i: (0, i, 0)),
            pl.BlockSpec((BR, D1), lambda i: (i, 0)),
            pl.BlockSpec((BR, NC), lambda i: (i, 0)),
            pl.BlockSpec((D1, 40), lambda i: (0, 0)),
            pl.BlockSpec((1, D1), lambda i: (0, 0)),
        ],
        out_specs=pl.BlockSpec((BR, D2P), lambda i: (i, 0)),
        out_shape=jax.ShapeDtypeStruct((N, D2P), jnp.float32),
    )(aggp, y1, degp, W2, b1)


def _tc_final(aggp2, y2, degp, b2p):
    def body(aggp_ref, y2_ref, degp_ref, b2_ref, out_ref):
        dinv = _dinv_of(degp_ref)
        agg = aggp_ref[0] + aggp_ref[1] + y2_ref[...]
        z = agg * dinv[:, None] + b2_ref[...]
        lane = lax.broadcasted_iota(jnp.int32, (BR, D2P), 1)
        zm = jnp.where(lane < 40, z, -1e30)
        m = jnp.max(zm, axis=1, keepdims=True)
        s = jnp.log(jnp.sum(jnp.exp(zm - m), axis=1, keepdims=True))
        out_ref[...] = z - m - s

    return pl.pallas_call(
        body,
        grid=(GRID,),
        in_specs=[
            pl.BlockSpec((NC, BR, D2P), lambda i: (0, i, 0)),
            pl.BlockSpec((BR, D2P), lambda i: (i, 0)),
            pl.BlockSpec((BR, NC), lambda i: (i, 0)),
            pl.BlockSpec((1, D2P), lambda i: (0, 0)),
        ],
        out_specs=pl.BlockSpec((BR, D2P), lambda i: (i, 0)),
        out_shape=jax.ShapeDtypeStruct((N, D2P), jnp.float32),
    )(aggp2, y2, degp, b2p)


def kernel(node_feature, adj_mat, W1, b1, W2, b2):
    src2d = adj_mat[0].reshape(E // CH, CH)
    dst2d = adj_mat[1].reshape(E // CH, CH)
    ones_ch = jnp.ones((CH, D1), jnp.float32)
    zeros_n16 = jnp.zeros((N, D1), jnp.float32)
    zeros_n48 = jnp.zeros((N, D2P), jnp.float32)
    b1r = b1.reshape(1, D1)
    b2p = jnp.concatenate([b2, jnp.zeros((D2P - 40,), jnp.float32)]).reshape(1, D2P)

    degp = _sc_degree(dst2d, ones_ch, zeros_n16)[:, :, 0].T
    y1 = _tc_y1(node_feature, W1, degp)
    aggp1 = _sc_agg(y1, src2d, dst2d, zeros_n16, D1)
    y2 = _tc_mid(aggp1, y1, degp, W2, b1r)
    aggp2 = _sc_agg(y2, src2d, dst2d, zeros_n48, D2P)
    out48 = _tc_final(aggp2, y2, degp, b2p)
    return out48[:, :40]


# speedup vs baseline: 32.8478x; 1.2641x over previous
import functools

import jax
import jax.numpy as jnp
from jax import lax
from jax.experimental import pallas as pl
from jax.experimental.pallas import tpu as pltpu
from jax.experimental.pallas import tpu_sc as plsc

N = 10000
E = 320000
D1 = 16
D2P = 48
NC = 2
NS = 16
CH = 125
CPT = E // CH // (NC * NS)
BR = 1000
GRID = N // BR


def _sc_mesh():
    return plsc.VectorSubcoreMesh(
        core_axis_name="c", subcore_axis_name="s", num_cores=NC, num_subcores=NS
    )


_SC_PARAMS = pltpu.CompilerParams(use_tc_tiling_on_sc=False)


def _sc_degree(dst2d, ones_ch, zeros_n16):
    @functools.partial(
        pl.kernel,
        out_type=jax.ShapeDtypeStruct((NC, N, D1), jnp.float32),
        mesh=_sc_mesh(),
        compiler_params=_SC_PARAMS,
        scratch_types=[
            pltpu.VMEM((CPT, CH), jnp.int32),
            pltpu.VMEM((CH, D1), jnp.float32),
            pltpu.VMEM_SHARED((N, D1), jnp.float32),
        ],
    )
    def body(dst_hbm, ones_hbm, zeros_hbm, out_hbm, idst, ones_v, acc):
        cid = lax.axis_index("c")
        sid = lax.axis_index("s")
        wid = cid * NS + sid
        pltpu.sync_copy(dst_hbm.at[pl.ds(wid * CPT, CPT)], idst)
        pltpu.sync_copy(ones_hbm, ones_v)
        @pl.when(sid < 10)
        def _():
            pltpu.sync_copy(
                zeros_hbm.at[pl.ds(sid * 1000, 1000)], acc.at[pl.ds(sid * 1000, 1000)]
            )
        plsc.subcore_barrier()

        def step(j, carry):
            pltpu.sync_copy(ones_v, acc.at[idst.at[j]], add=True)
            return carry

        lax.fori_loop(0, CPT, step, 0)
        plsc.subcore_barrier()

        @pl.when(sid < 10)
        def _():
            pltpu.sync_copy(
                acc.at[pl.ds(sid * 1000, 1000)],
                out_hbm.at[cid].at[pl.ds(sid * 1000, 1000)],
            )

    return body(dst2d, ones_ch, zeros_n16)


def _sc_agg(y, src2d, dst2d, zeros_nd, d):
    rpt = 1000

    @functools.partial(
        pl.kernel,
        out_type=jax.ShapeDtypeStruct((NC, N, d), jnp.float32),
        mesh=_sc_mesh(),
        compiler_params=_SC_PARAMS,
        scratch_types=[
            pltpu.VMEM((CPT, CH), jnp.int32),
            pltpu.VMEM((CPT, CH), jnp.int32),
            pltpu.VMEM((CH, d), jnp.float32),
            pltpu.VMEM((CH, d), jnp.float32),
            pltpu.VMEM_SHARED((N, d), jnp.float32),
            pltpu.SemaphoreType.DMA,
            pltpu.SemaphoreType.DMA,
        ],
    )
    def body(
        y_hbm, src_hbm, dst_hbm, zeros_hbm, out_hbm, isrc, idst, buf0, buf1, acc, sem0, sem1
    ):
        cid = lax.axis_index("c")
        sid = lax.axis_index("s")
        wid = cid * NS + sid
        pltpu.sync_copy(src_hbm.at[pl.ds(wid * CPT, CPT)], isrc)
        pltpu.sync_copy(dst_hbm.at[pl.ds(wid * CPT, CPT)], idst)

        @pl.when(sid < 10)
        def _():
            pltpu.sync_copy(
                zeros_hbm.at[pl.ds(sid * rpt, rpt)], acc.at[pl.ds(sid * rpt, rpt)]
            )
        plsc.subcore_barrier()

        pltpu.async_copy(y_hbm.at[isrc.at[0]], buf0, sem0)

        def step(k, carry):
            j0 = 2 * k
            j1 = j0 + 1
            pltpu.async_copy(y_hbm.at[isrc.at[j1]], buf1, sem1)
            pltpu.make_async_copy(y_hbm.at[isrc.at[j0]], buf0, sem0).wait()
            pltpu.sync_copy(buf0, acc.at[idst.at[j0]], add=True)

            @pl.when(k < CPT // 2 - 1)
            def _():
                pltpu.async_copy(y_hbm.at[isrc.at[j0 + 2]], buf0, sem0)

            pltpu.make_async_copy(y_hbm.at[isrc.at[j1]], buf1, sem1).wait()
            pltpu.sync_copy(buf1, acc.at[idst.at[j1]], add=True)
            return carry

        lax.fori_loop(0, CPT // 2, step, 0)
        plsc.subcore_barrier()

        @pl.when(sid < 10)
        def _():
            pltpu.sync_copy(
                acc.at[pl.ds(sid * rpt, rpt)],
                out_hbm.at[cid].at[pl.ds(sid * rpt, rpt)],
            )

    return body(y, src2d, dst2d, zeros_nd)


def _dinv_of(degp_ref):
    deg = jnp.sum(degp_ref[...], axis=1) + 1.0
    return lax.rsqrt(deg)


def _tc_y1(x, W1, degp):
    def body(x_ref, w_ref, degp_ref, y1_ref):
        dinv = _dinv_of(degp_ref)
        xw = jnp.dot(x_ref[...], w_ref[...], preferred_element_type=jnp.float32)
        y1_ref[...] = xw * dinv[:, None]

    return pl.pallas_call(
        body,
        grid=(GRID,),
        in_specs=[
            pl.BlockSpec((BR, 128), lambda i: (i, 0)),
            pl.BlockSpec((128, D1), lambda i: (0, 0)),
            pl.BlockSpec((BR, NC), lambda i: (i, 0)),
        ],
        out_specs=pl.BlockSpec((BR, D1), lambda i: (i, 0)),
        out_shape=jax.ShapeDtypeStruct((N, D1), jnp.float32),
    )(x, W1, degp)


def _tc_mid(aggp, y1, degp, W2, b1):
    def body(aggp_ref, y1_ref, degp_ref, w_ref, b1_ref, y2_ref):
        dinv = _dinv_of(degp_ref)
        agg = aggp_ref[0] + aggp_ref[1] + y1_ref[...]
        pre = agg * dinv[:, None] + b1_ref[...]
        h = jnp.where(pre > 0, pre, jnp.exp(jnp.minimum(pre, 0.0)) - 1.0)
        hw = jnp.dot(h, w_ref[...], preferred_element_type=jnp.float32)
        y2 = hw * dinv[:, None]
        y2_ref[...] = jnp.concatenate(
            [y2, jnp.zeros((BR, D2P - 40), jnp.float32)], axis=1
        )

    return pl.pallas_call(
        body,
        grid=(GRID,),
        in_specs=[
            pl.BlockSpec((NC, BR, D1), lambda i: (0, i, 0)),
            pl.BlockSpec((BR, D1), lambda i: (i, 0)),
            pl.BlockSpec((BR, NC), lambda i: (i, 0)),
            pl.BlockSpec((D1, 40), lambda i: (0, 0)),
            pl.BlockSpec((1, D1), lambda i: (0, 0)),
        ],
        out_specs=pl.BlockSpec((BR, D2P), lambda i: (i, 0)),
        out_shape=jax.ShapeDtypeStruct((N, D2P), jnp.float32),
    )(aggp, y1, degp, W2, b1)


def _tc_final(aggp2, y2, degp, b2p):
    def body(aggp_ref, y2_ref, degp_ref, b2_ref, out_ref):
        dinv = _dinv_of(degp_ref)
        agg = aggp_ref[0] + aggp_ref[1] + y2_ref[...]
        z = agg * dinv[:, None] + b2_ref[...]
        lane = lax.broadcasted_iota(jnp.int32, (BR, D2P), 1)
        zm = jnp.where(lane < 40, z, -1e30)
        m = jnp.max(zm, axis=1, keepdims=True)
        s = jnp.log(jnp.sum(jnp.exp(zm - m), axis=1, keepdims=True))
        out_ref[...] = (z - m - s)[:, :40]

    return pl.pallas_call(
        body,
        grid=(GRID,),
        in_specs=[
            pl.BlockSpec((NC, BR, D2P), lambda i: (0, i, 0)),
            pl.BlockSpec((BR, D2P), lambda i: (i, 0)),
            pl.BlockSpec((BR, NC), lambda i: (i, 0)),
            pl.BlockSpec((1, D2P), lambda i: (0, 0)),
        ],
        out_specs=pl.BlockSpec((BR, 40), lambda i: (i, 0)),
        out_shape=jax.ShapeDtypeStruct((N, 40), jnp.float32),
    )(aggp2, y2, degp, b2p)


def kernel(node_feature, adj_mat, W1, b1, W2, b2):
    src2d = adj_mat[0].reshape(E // CH, CH)
    dst2d = adj_mat[1].reshape(E // CH, CH)
    ones_ch = jnp.ones((CH, D1), jnp.float32)
    zeros_n16 = jnp.zeros((N, D1), jnp.float32)
    zeros_n48 = jnp.zeros((N, D2P), jnp.float32)
    b1r = b1.reshape(1, D1)
    b2p = jnp.concatenate([b2, jnp.zeros((D2P - 40,), jnp.float32)]).reshape(1, D2P)

    degp = _sc_degree(dst2d, ones_ch, zeros_n16)[:, :, 0].T
    y1 = _tc_y1(node_feature, W1, degp)
    aggp1 = _sc_agg(y1, src2d, dst2d, zeros_n16, D1)
    y2 = _tc_mid(aggp1, y1, degp, W2, b1r)
    aggp2 = _sc_agg(y2, src2d, dst2d, zeros_n48, D2P)
    return _tc_final(aggp2, y2, degp, b2p)


# speedup vs baseline: 41.4308x; 1.2613x over previous
import functools

import jax
import jax.numpy as jnp
from jax import lax
from jax.experimental import pallas as pl
from jax.experimental.pallas import tpu as pltpu
from jax.experimental.pallas import tpu_sc as plsc

N = 10000
E = 320000
D1 = 16
D2P = 48
NC = 2
NS = 16
CH = 125
CPT = E // CH // (NC * NS)
BR = 1000
GRID = N // BR


def _sc_mesh():
    return plsc.VectorSubcoreMesh(
        core_axis_name="c", subcore_axis_name="s", num_cores=NC, num_subcores=NS
    )


_SC_PARAMS = pltpu.CompilerParams(use_tc_tiling_on_sc=False)


def _sc_degree(dst2d, ones_ch, zeros_n16):
    @functools.partial(
        pl.kernel,
        out_type=jax.ShapeDtypeStruct((NC, N, D1), jnp.float32),
        mesh=_sc_mesh(),
        compiler_params=_SC_PARAMS,
        scratch_types=[
            pltpu.VMEM((CPT, CH), jnp.int32),
            pltpu.VMEM((CH, D1), jnp.float32),
            pltpu.VMEM_SHARED((N, D1), jnp.float32),
        ],
    )
    def body(dst_hbm, ones_hbm, zeros_hbm, out_hbm, idst, ones_v, acc):
        cid = lax.axis_index("c")
        sid = lax.axis_index("s")
        wid = cid * NS + sid
        pltpu.sync_copy(dst_hbm.at[pl.ds(wid * CPT, CPT)], idst)
        pltpu.sync_copy(ones_hbm, ones_v)
        @pl.when(sid < 10)
        def _():
            pltpu.sync_copy(
                zeros_hbm.at[pl.ds(sid * 1000, 1000)], acc.at[pl.ds(sid * 1000, 1000)]
            )
        plsc.subcore_barrier()

        def step(j, carry):
            pltpu.sync_copy(ones_v, acc.at[idst.at[j]], add=True)
            return carry

        lax.fori_loop(0, CPT, step, 0)
        plsc.subcore_barrier()

        @pl.when(sid < 10)
        def _():
            pltpu.sync_copy(
                acc.at[pl.ds(sid * 1000, 1000)],
                out_hbm.at[cid].at[pl.ds(sid * 1000, 1000)],
            )

    return body(dst2d, ones_ch, zeros_n16)


def _sc_agg(y, src2d, dst2d, zeros_nd, d):
    rpt = 1000

    @functools.partial(
        pl.kernel,
        out_type=jax.ShapeDtypeStruct((NC, N, d), jnp.float32),
        mesh=_sc_mesh(),
        compiler_params=_SC_PARAMS,
        scratch_types=[
            pltpu.VMEM((CPT, CH), jnp.int32),
            pltpu.VMEM((CPT, CH), jnp.int32),
            pltpu.VMEM((CH, d), jnp.float32),
            pltpu.VMEM((CH, d), jnp.float32),
            pltpu.VMEM_SHARED((N, d), jnp.float32),
            pltpu.SemaphoreType.DMA,
            pltpu.SemaphoreType.DMA,
        ],
    )
    def body(
        y_hbm, src_hbm, dst_hbm, zeros_hbm, out_hbm, isrc, idst, buf0, buf1, acc, sem0, sem1
    ):
        cid = lax.axis_index("c")
        sid = lax.axis_index("s")
        wid = cid * NS + sid
        pltpu.sync_copy(src_hbm.at[pl.ds(wid * CPT, CPT)], isrc)
        pltpu.sync_copy(dst_hbm.at[pl.ds(wid * CPT, CPT)], idst)

        @pl.when(sid < 10)
        def _():
            pltpu.sync_copy(
                zeros_hbm.at[pl.ds(sid * rpt, rpt)], acc.at[pl.ds(sid * rpt, rpt)]
            )
        plsc.subcore_barrier()

        pltpu.async_copy(y_hbm.at[isrc.at[0]], buf0, sem0)

        def step(k, carry):
            j0 = 2 * k
            j1 = j0 + 1
            pltpu.async_copy(y_hbm.at[isrc.at[j1]], buf1, sem1)
            pltpu.make_async_copy(y_hbm.at[isrc.at[j0]], buf0, sem0).wait()
            pltpu.sync_copy(buf0, acc.at[idst.at[j0]], add=True)

            @pl.when(k < CPT // 2 - 1)
            def _():
                pltpu.async_copy(y_hbm.at[isrc.at[j0 + 2]], buf0, sem0)

            pltpu.make_async_copy(y_hbm.at[isrc.at[j1]], buf1, sem1).wait()
            pltpu.sync_copy(buf1, acc.at[idst.at[j1]], add=True)
            return carry

        lax.fori_loop(0, CPT // 2, step, 0)
        plsc.subcore_barrier()

        @pl.when(sid < 10)
        def _():
            pltpu.sync_copy(
                acc.at[pl.ds(sid * rpt, rpt)],
                out_hbm.at[cid].at[pl.ds(sid * rpt, rpt)],
            )

    return body(y, src2d, dst2d, zeros_nd)


def _dinv_of(degp_ref):
    deg = degp_ref[0] + degp_ref[1] + 1.0
    return lax.rsqrt(deg)


def _tc_y1(x, W1, degp):
    def body(x_ref, w_ref, degp_ref, y1_ref):
        dinv = _dinv_of(degp_ref)
        xw = jnp.dot(x_ref[...], w_ref[...], preferred_element_type=jnp.float32)
        y1_ref[...] = xw * dinv

    return pl.pallas_call(
        body,
        out_shape=jax.ShapeDtypeStruct((N, D1), jnp.float32),
    )(x, W1, degp)


def _tc_mid(aggp, y1, degp, W2, b1):
    def body(aggp_ref, y1_ref, degp_ref, w_ref, b1_ref, y2_ref):
        dinv = _dinv_of(degp_ref)
        agg = aggp_ref[0] + aggp_ref[1] + y1_ref[...]
        pre = agg * dinv + b1_ref[...]
        h = jnp.where(pre > 0, pre, jnp.exp(jnp.minimum(pre, 0.0)) - 1.0)
        hw = jnp.dot(h, w_ref[...], preferred_element_type=jnp.float32)
        y2 = hw * dinv[:, :1]
        y2_ref[...] = jnp.concatenate(
            [y2, jnp.zeros((N, D2P - 40), jnp.float32)], axis=1
        )

    return pl.pallas_call(
        body,
        out_shape=jax.ShapeDtypeStruct((N, D2P), jnp.float32),
    )(aggp, y1, degp, W2, b1)


def _tc_final(aggp2, y2, degp, b2p):
    def body(aggp_ref, y2_ref, degp_ref, b2_ref, out_ref):
        dinv = _dinv_of(degp_ref)
        agg = aggp_ref[0] + aggp_ref[1] + y2_ref[...]
        z = agg * dinv[:, :1] + b2_ref[...]
        lane = lax.broadcasted_iota(jnp.int32, (N, D2P), 1)
        zm = jnp.where(lane < 40, z, -1e30)
        m = jnp.max(zm, axis=1, keepdims=True)
        s = jnp.log(jnp.sum(jnp.exp(zm - m), axis=1, keepdims=True))
        out_ref[...] = (z - m - s)[:, :40]

    return pl.pallas_call(
        body,
        out_shape=jax.ShapeDtypeStruct((N, 40), jnp.float32),
    )(aggp2, y2, degp, b2p)


def kernel(node_feature, adj_mat, W1, b1, W2, b2):
    src2d = adj_mat[0].reshape(E // CH, CH)
    dst2d = adj_mat[1].reshape(E // CH, CH)
    ones_ch = jnp.ones((CH, D1), jnp.float32)
    zeros_n16 = jnp.zeros((N, D1), jnp.float32)
    zeros_n48 = jnp.zeros((N, D2P), jnp.float32)
    b1r = b1.reshape(1, D1)
    b2p = jnp.concatenate([b2, jnp.zeros((D2P - 40,), jnp.float32)]).reshape(1, D2P)

    degp = _sc_degree(dst2d, ones_ch, zeros_n16)
    y1 = _tc_y1(node_feature, W1, degp)
    aggp1 = _sc_agg(y1, src2d, dst2d, zeros_n16, D1)
    y2 = _tc_mid(aggp1, y1, degp, W2, b1r)
    aggp2 = _sc_agg(y2, src2d, dst2d, zeros_n48, D2P)
    return _tc_final(aggp2, y2, degp, b2p)


# speedup vs baseline: 50.0761x; 1.2087x over previous
import functools

import jax
import jax.numpy as jnp
from jax import lax
from jax.experimental import pallas as pl
from jax.experimental.pallas import tpu as pltpu
from jax.experimental.pallas import tpu_sc as plsc

N = 10000
E = 320000
D1 = 16
D2P = 48
NC = 2
NS = 16
CH = 125
CPT = E // CH // (NC * NS)
BR = 1000
GRID = N // BR
RING = 8


def _sc_mesh():
    return plsc.VectorSubcoreMesh(
        core_axis_name="c", subcore_axis_name="s", num_cores=NC, num_subcores=NS
    )


_SC_PARAMS = pltpu.CompilerParams(use_tc_tiling_on_sc=False)


def _sc_degree(dst2d, ones_ch, zeros_n16):
    @functools.partial(
        pl.kernel,
        out_type=jax.ShapeDtypeStruct((NC, N, D1), jnp.float32),
        mesh=_sc_mesh(),
        compiler_params=_SC_PARAMS,
        scratch_types=[
            pltpu.VMEM((CPT, CH), jnp.int32),
            pltpu.VMEM((CH, D1), jnp.float32),
            pltpu.VMEM_SHARED((N, D1), jnp.float32),
            pltpu.SemaphoreType.DMA,
        ],
    )
    def body(dst_hbm, ones_hbm, zeros_hbm, out_hbm, idst, ones_v, acc, sem):
        cid = lax.axis_index("c")
        sid = lax.axis_index("s")
        wid = cid * NS + sid
        pltpu.sync_copy(dst_hbm.at[pl.ds(wid * CPT, CPT)], idst)
        pltpu.sync_copy(ones_hbm, ones_v)
        @pl.when(sid < 10)
        def _():
            pltpu.sync_copy(
                zeros_hbm.at[pl.ds(sid * 1000, 1000)], acc.at[pl.ds(sid * 1000, 1000)]
            )
        plsc.subcore_barrier()

        def step(j, carry):
            pltpu.async_copy(ones_v, acc.at[idst.at[j]], sem, add=True)
            return carry

        lax.fori_loop(0, CPT, step, 0)

        def drain(j, carry):
            pltpu.make_async_copy(ones_v, acc.at[idst.at[j]], sem).wait()
            return carry

        lax.fori_loop(0, CPT, drain, 0)
        plsc.subcore_barrier()

        @pl.when(sid < 10)
        def _():
            pltpu.sync_copy(
                acc.at[pl.ds(sid * 1000, 1000)],
                out_hbm.at[cid].at[pl.ds(sid * 1000, 1000)],
            )

    return body(dst2d, ones_ch, zeros_n16)


def _sc_agg(y, src2d, dst2d, zeros_nd, d):
    rpt = 1000

    @functools.partial(
        pl.kernel,
        out_type=jax.ShapeDtypeStruct((NC, N, d), jnp.float32),
        mesh=_sc_mesh(),
        compiler_params=_SC_PARAMS,
        scratch_types=[
            pltpu.VMEM((CPT, CH), jnp.int32),
            pltpu.VMEM((CPT, CH), jnp.int32),
            [pltpu.VMEM((CH, d), jnp.float32) for _ in range(RING)],
            [pltpu.SemaphoreType.DMA for _ in range(RING)],
            [pltpu.SemaphoreType.DMA for _ in range(RING)],
            pltpu.VMEM_SHARED((N, d), jnp.float32),
        ],
    )
    def body(
        y_hbm, src_hbm, dst_hbm, zeros_hbm, out_hbm, isrc, idst, bufs, gsems, ssems, acc
    ):
        cid = lax.axis_index("c")
        sid = lax.axis_index("s")
        wid = cid * NS + sid
        pltpu.sync_copy(src_hbm.at[pl.ds(wid * CPT, CPT)], isrc)
        pltpu.sync_copy(dst_hbm.at[pl.ds(wid * CPT, CPT)], idst)

        @pl.when(sid < 10)
        def _():
            pltpu.sync_copy(
                zeros_hbm.at[pl.ds(sid * rpt, rpt)], acc.at[pl.ds(sid * rpt, rpt)]
            )
        plsc.subcore_barrier()

        for b in range(RING):
            pltpu.async_copy(y_hbm.at[isrc.at[b]], bufs[b], gsems[b])

        def step(k, carry):
            base = k * RING
            for b in range(RING):
                j = base + b
                pltpu.make_async_copy(y_hbm.at[isrc.at[j]], bufs[b], gsems[b]).wait()
                pltpu.async_copy(bufs[b], acc.at[idst.at[j]], ssems[b], add=True)
            for b in range(RING):
                j = base + b

                @pl.when(j + RING < CPT)
                def _():
                    pltpu.make_async_copy(bufs[b], acc.at[idst.at[j]], ssems[b]).wait()
                    pltpu.async_copy(y_hbm.at[isrc.at[j + RING]], bufs[b], gsems[b])

            return carry

        lax.fori_loop(0, CPT // RING, step, 0)
        for b in range(RING):
            pltpu.make_async_copy(bufs[b], acc.at[idst.at[CPT - RING + b]], ssems[b]).wait()
        plsc.subcore_barrier()

        @pl.when(sid < 10)
        def _():
            pltpu.sync_copy(
                acc.at[pl.ds(sid * rpt, rpt)],
                out_hbm.at[cid].at[pl.ds(sid * rpt, rpt)],
            )

    return body(y, src2d, dst2d, zeros_nd)


def _dinv_of(degp_ref):
    deg = degp_ref[0] + degp_ref[1] + 1.0
    return lax.rsqrt(deg)


def _tc_y1(x, W1, degp):
    def body(x_ref, w_ref, degp_ref, y1_ref):
        dinv = _dinv_of(degp_ref)
        xw = jnp.dot(x_ref[...], w_ref[...], preferred_element_type=jnp.float32)
        y1_ref[...] = xw * dinv

    return pl.pallas_call(
        body,
        out_shape=jax.ShapeDtypeStruct((N, D1), jnp.float32),
    )(x, W1, degp)


def _tc_mid(aggp, y1, degp, W2, b1):
    def body(aggp_ref, y1_ref, degp_ref, w_ref, b1_ref, y2_ref):
        dinv = _dinv_of(degp_ref)
        agg = aggp_ref[0] + aggp_ref[1] + y1_ref[...]
        pre = agg * dinv + b1_ref[...]
        h = jnp.where(pre > 0, pre, jnp.exp(jnp.minimum(pre, 0.0)) - 1.0)
        hw = jnp.dot(h, w_ref[...], preferred_element_type=jnp.float32)
        y2 = hw * dinv[:, :1]
        y2_ref[...] = jnp.concatenate(
            [y2, jnp.zeros((N, D2P - 40), jnp.float32)], axis=1
        )

    return pl.pallas_call(
        body,
        out_shape=jax.ShapeDtypeStruct((N, D2P), jnp.float32),
    )(aggp, y1, degp, W2, b1)


def _tc_final(aggp2, y2, degp, b2p):
    def body(aggp_ref, y2_ref, degp_ref, b2_ref, out_ref):
        dinv = _dinv_of(degp_ref)
        agg = aggp_ref[0] + aggp_ref[1] + y2_ref[...]
        z = agg * dinv[:, :1] + b2_ref[...]
        lane = lax.broadcasted_iota(jnp.int32, (N, D2P), 1)
        zm = jnp.where(lane < 40, z, -1e30)
        m = jnp.max(zm, axis=1, keepdims=True)
        s = jnp.log(jnp.sum(jnp.exp(zm - m), axis=1, keepdims=True))
        out_ref[...] = (z - m - s)[:, :40]

    return pl.pallas_call(
        body,
        out_shape=jax.ShapeDtypeStruct((N, 40), jnp.float32),
    )(aggp2, y2, degp, b2p)


def kernel(node_feature, adj_mat, W1, b1, W2, b2):
    src2d = adj_mat[0].reshape(E // CH, CH)
    dst2d = adj_mat[1].reshape(E // CH, CH)
    ones_ch = jnp.ones((CH, D1), jnp.float32)
    zeros_n16 = jnp.zeros((N, D1), jnp.float32)
    zeros_n48 = jnp.zeros((N, D2P), jnp.float32)
    b1r = b1.reshape(1, D1)
    b2p = jnp.concatenate([b2, jnp.zeros((D2P - 40,), jnp.float32)]).reshape(1, D2P)

    degp = _sc_degree(dst2d, ones_ch, zeros_n16)
    y1 = _tc_y1(node_feature, W1, degp)
    aggp1 = _sc_agg(y1, src2d, dst2d, zeros_n16, D1)
    y2 = _tc_mid(aggp1, y1, degp, W2, b1r)
    aggp2 = _sc_agg(y2, src2d, dst2d, zeros_n48, D2P)
    return _tc_final(aggp2, y2, degp, b2p)


# speedup vs baseline: 51.4833x; 1.0281x over previous
import functools

import jax
import jax.numpy as jnp
from jax import lax
from jax.experimental import pallas as pl
from jax.experimental.pallas import tpu as pltpu
from jax.experimental.pallas import tpu_sc as plsc

N = 10000
E = 320000
D1 = 16
D2P = 40
NC = 2
NS = 16
CH = 125
CPT = E // CH // (NC * NS)
BR = 1000
GRID = N // BR
RING = 8


def _sc_mesh():
    return plsc.VectorSubcoreMesh(
        core_axis_name="c", subcore_axis_name="s", num_cores=NC, num_subcores=NS
    )


_SC_PARAMS = pltpu.CompilerParams(use_tc_tiling_on_sc=False)


def _sc_degree(dst2d, ones_ch, zeros_n16):
    @functools.partial(
        pl.kernel,
        out_type=jax.ShapeDtypeStruct((NC, N, D1), jnp.float32),
        mesh=_sc_mesh(),
        compiler_params=_SC_PARAMS,
        scratch_types=[
            pltpu.VMEM((CPT, CH), jnp.int32),
            pltpu.VMEM((CH, D1), jnp.float32),
            pltpu.VMEM_SHARED((N, D1), jnp.float32),
            pltpu.SemaphoreType.DMA,
        ],
    )
    def body(dst_hbm, ones_hbm, zeros_hbm, out_hbm, idst, ones_v, acc, sem):
        cid = lax.axis_index("c")
        sid = lax.axis_index("s")
        wid = cid * NS + sid
        pltpu.sync_copy(dst_hbm.at[pl.ds(wid * CPT, CPT)], idst)
        pltpu.sync_copy(ones_hbm, ones_v)
        @pl.when(sid < 10)
        def _():
            pltpu.sync_copy(
                zeros_hbm.at[pl.ds(sid * 1000, 1000)], acc.at[pl.ds(sid * 1000, 1000)]
            )
        plsc.subcore_barrier()

        def step(j, carry):
            pltpu.async_copy(ones_v, acc.at[idst.at[j]], sem, add=True)
            return carry

        lax.fori_loop(0, CPT, step, 0)

        def drain(j, carry):
            pltpu.make_async_copy(ones_v, acc.at[idst.at[j]], sem).wait()
            return carry

        lax.fori_loop(0, CPT, drain, 0)
        plsc.subcore_barrier()

        @pl.when(sid < 10)
        def _():
            pltpu.sync_copy(
                acc.at[pl.ds(sid * 1000, 1000)],
                out_hbm.at[cid].at[pl.ds(sid * 1000, 1000)],
            )

    return body(dst2d, ones_ch, zeros_n16)


def _sc_agg(y, src2d, dst2d, zeros_nd, d):
    rpt = 1000

    @functools.partial(
        pl.kernel,
        out_type=jax.ShapeDtypeStruct((NC, N, d), jnp.float32),
        mesh=_sc_mesh(),
        compiler_params=_SC_PARAMS,
        scratch_types=[
            pltpu.VMEM((CPT, CH), jnp.int32),
            pltpu.VMEM((CPT, CH), jnp.int32),
            [pltpu.VMEM((CH, d), jnp.float32) for _ in range(RING)],
            [pltpu.SemaphoreType.DMA for _ in range(RING)],
            [pltpu.SemaphoreType.DMA for _ in range(RING)],
            pltpu.VMEM_SHARED((N, d), jnp.float32),
        ],
    )
    def body(
        y_hbm, src_hbm, dst_hbm, zeros_hbm, out_hbm, isrc, idst, bufs, gsems, ssems, acc
    ):
        cid = lax.axis_index("c")
        sid = lax.axis_index("s")
        wid = cid * NS + sid
        pltpu.sync_copy(src_hbm.at[pl.ds(wid * CPT, CPT)], isrc)
        pltpu.sync_copy(dst_hbm.at[pl.ds(wid * CPT, CPT)], idst)

        @pl.when(sid < 10)
        def _():
            pltpu.sync_copy(
                zeros_hbm.at[pl.ds(sid * rpt, rpt)], acc.at[pl.ds(sid * rpt, rpt)]
            )
        plsc.subcore_barrier()

        for b in range(RING):
            pltpu.async_copy(y_hbm.at[isrc.at[b]], bufs[b], gsems[b])

        def step(k, carry):
            base = k * RING
            for b in range(RING):
                j = base + b
                pltpu.make_async_copy(y_hbm.at[isrc.at[j]], bufs[b], gsems[b]).wait()
                pltpu.async_copy(bufs[b], acc.at[idst.at[j]], ssems[b], add=True)
            for b in range(RING):
                j = base + b

                @pl.when(j + RING < CPT)
                def _():
                    pltpu.make_async_copy(bufs[b], acc.at[idst.at[j]], ssems[b]).wait()
                    pltpu.async_copy(y_hbm.at[isrc.at[j + RING]], bufs[b], gsems[b])

            return carry

        lax.fori_loop(0, CPT // RING, step, 0)
        for b in range(RING):
            pltpu.make_async_copy(bufs[b], acc.at[idst.at[CPT - RING + b]], ssems[b]).wait()
        plsc.subcore_barrier()

        @pl.when(sid < 10)
        def _():
            pltpu.sync_copy(
                acc.at[pl.ds(sid * rpt, rpt)],
                out_hbm.at[cid].at[pl.ds(sid * rpt, rpt)],
            )

    return body(y, src2d, dst2d, zeros_nd)


def _dinv_of(degp_ref):
    deg = degp_ref[0] + degp_ref[1] + 1.0
    return lax.rsqrt(deg)


def _tc_y1(x, W1, degp):
    def body(x_ref, w_ref, degp_ref, y1_ref):
        dinv = _dinv_of(degp_ref)
        xw = jnp.dot(x_ref[...], w_ref[...], preferred_element_type=jnp.float32)
        y1_ref[...] = xw * dinv

    return pl.pallas_call(
        body,
        out_shape=jax.ShapeDtypeStruct((N, D1), jnp.float32),
    )(x, W1, degp)


def _tc_mid(aggp, y1, degp, W2, b1):
    def body(aggp_ref, y1_ref, degp_ref, w_ref, b1_ref, y2_ref):
        dinv = _dinv_of(degp_ref)
        agg = aggp_ref[0] + aggp_ref[1] + y1_ref[...]
        pre = agg * dinv + b1_ref[...]
        h = jnp.where(pre > 0, pre, jnp.exp(jnp.minimum(pre, 0.0)) - 1.0)
        hw = jnp.dot(h, w_ref[...], preferred_element_type=jnp.float32)
        y2_ref[...] = hw * dinv[:, :1]

    return pl.pallas_call(
        body,
        out_shape=jax.ShapeDtypeStruct((N, D2P), jnp.float32),
    )(aggp, y1, degp, W2, b1)


def _tc_final(aggp2, y2, degp, b2p):
    def body(aggp_ref, y2_ref, degp_ref, b2_ref, out_ref):
        dinv = _dinv_of(degp_ref)
        agg = aggp_ref[0] + aggp_ref[1] + y2_ref[...]
        z = agg * dinv[:, :1] + b2_ref[...]
        lane = lax.broadcasted_iota(jnp.int32, (N, D2P), 1)
        zm = jnp.where(lane < 40, z, -1e30)
        m = jnp.max(zm, axis=1, keepdims=True)
        s = jnp.log(jnp.sum(jnp.exp(zm - m), axis=1, keepdims=True))
        out_ref[...] = (z - m - s)[:, :40]

    return pl.pallas_call(
        body,
        out_shape=jax.ShapeDtypeStruct((N, 40), jnp.float32),
    )(aggp2, y2, degp, b2p)


def kernel(node_feature, adj_mat, W1, b1, W2, b2):
    src2d = adj_mat[0].reshape(E // CH, CH)
    dst2d = adj_mat[1].reshape(E // CH, CH)
    ones_ch = jnp.ones((CH, D1), jnp.float32)
    zeros_n16 = jnp.zeros((N, D1), jnp.float32)
    zeros_n48 = jnp.zeros((N, D2P), jnp.float32)
    b1r = b1.reshape(1, D1)
    b2p = jnp.concatenate([b2, jnp.zeros((D2P - 40,), jnp.float32)]).reshape(1, D2P)

    degp = _sc_degree(dst2d, ones_ch, zeros_n16)
    y1 = _tc_y1(node_feature, W1, degp)
    aggp1 = _sc_agg(y1, src2d, dst2d, zeros_n16, D1)
    y2 = _tc_mid(aggp1, y1, degp, W2, b1r)
    aggp2 = _sc_agg(y2, src2d, dst2d, zeros_n48, D2P)
    return _tc_final(aggp2, y2, degp, b2p)


# speedup vs baseline: 51.7408x; 1.0050x over previous
import functools

import jax
import jax.numpy as jnp
from jax import lax
from jax.experimental import pallas as pl
from jax.experimental.pallas import tpu as pltpu
from jax.experimental.pallas import tpu_sc as plsc

N = 10000
E = 320000
D1 = 16
D2P = 40
NC = 2
NS = 16
CH = 125
CPT = E // CH // (NC * NS)
BR = 1000
GRID = N // BR
RING = 8


def _sc_mesh():
    return plsc.VectorSubcoreMesh(
        core_axis_name="c", subcore_axis_name="s", num_cores=NC, num_subcores=NS
    )


_SC_PARAMS = pltpu.CompilerParams(use_tc_tiling_on_sc=False)


def _sc_degree(dst2d, ones_ch, zeros_n16):
    @functools.partial(
        pl.kernel,
        out_type=jax.ShapeDtypeStruct((NC, N, D1), jnp.float32),
        mesh=_sc_mesh(),
        compiler_params=_SC_PARAMS,
        scratch_types=[
            pltpu.VMEM((CPT, CH), jnp.int32),
            pltpu.VMEM((CH, D1), jnp.float32),
            pltpu.VMEM_SHARED((N, D1), jnp.float32),
            pltpu.SemaphoreType.DMA,
        ],
    )
    def body(dst_hbm, ones_hbm, zeros_hbm, out_hbm, idst, ones_v, acc, sem):
        cid = lax.axis_index("c")
        sid = lax.axis_index("s")
        wid = cid * NS + sid
        pltpu.sync_copy(dst_hbm.at[pl.ds(wid * CPT, CPT)], idst)
        pltpu.sync_copy(ones_hbm, ones_v)
        @pl.when(sid < 10)
        def _():
            pltpu.sync_copy(
                zeros_hbm.at[pl.ds(sid * 1000, 1000)], acc.at[pl.ds(sid * 1000, 1000)]
            )
        plsc.subcore_barrier()

        def step(j, carry):
            pltpu.async_copy(ones_v, acc.at[idst.at[j]], sem, add=True)
            return carry

        lax.fori_loop(0, CPT, step, 0)

        def drain(j, carry):
            pltpu.make_async_copy(ones_v, acc.at[idst.at[j]], sem).wait()
            return carry

        lax.fori_loop(0, CPT, drain, 0)
        plsc.subcore_barrier()

        @pl.when(sid < 10)
        def _():
            pltpu.sync_copy(
                acc.at[pl.ds(sid * 1000, 1000)],
                out_hbm.at[cid].at[pl.ds(sid * 1000, 1000)],
            )

    return body(dst2d, ones_ch, zeros_n16)


def _sc_agg(y, src2d, dst2d, zeros_nd, d):
    rpt = 1000

    @functools.partial(
        pl.kernel,
        out_type=jax.ShapeDtypeStruct((NC, N, d), jnp.float32),
        mesh=_sc_mesh(),
        compiler_params=_SC_PARAMS,
        scratch_types=[
            pltpu.VMEM((CPT, CH), jnp.int32),
            pltpu.VMEM((CPT, CH), jnp.int32),
            [pltpu.VMEM((CH, d), jnp.float32) for _ in range(RING)],
            [pltpu.SemaphoreType.DMA for _ in range(RING)],
            [pltpu.SemaphoreType.DMA for _ in range(RING)],
            pltpu.VMEM_SHARED((N, d), jnp.float32),
        ],
    )
    def body(
        y_hbm, src_hbm, dst_hbm, zeros_hbm, out_hbm, isrc, idst, bufs, gsems, ssems, acc
    ):
        cid = lax.axis_index("c")
        sid = lax.axis_index("s")
        wid = cid * NS + sid
        pltpu.sync_copy(src_hbm.at[pl.ds(wid * CPT, CPT)], isrc)
        pltpu.sync_copy(dst_hbm.at[pl.ds(wid * CPT, CPT)], idst)

        @pl.when(sid < 10)
        def _():
            pltpu.sync_copy(
                zeros_hbm.at[pl.ds(sid * rpt, rpt)], acc.at[pl.ds(sid * rpt, rpt)]
            )
        plsc.subcore_barrier()

        for b in range(RING):
            pltpu.async_copy(y_hbm.at[isrc.at[b]], bufs[b], gsems[b])

        def step(k, carry):
            base = k * RING
            for b in range(RING):
                j = base + b
                pltpu.make_async_copy(y_hbm.at[isrc.at[j]], bufs[b], gsems[b]).wait()
                pltpu.async_copy(bufs[b], acc.at[idst.at[j]], ssems[b], add=True)
            for b in range(RING):
                j = base + b

                @pl.when(j + RING < CPT)
                def _():
                    pltpu.make_async_copy(bufs[b], acc.at[idst.at[j]], ssems[b]).wait()
                    pltpu.async_copy(y_hbm.at[isrc.at[j + RING]], bufs[b], gsems[b])

            return carry

        lax.fori_loop(0, CPT // RING, step, 0)
        for b in range(RING):
            pltpu.make_async_copy(bufs[b], acc.at[idst.at[CPT - RING + b]], ssems[b]).wait()
        plsc.subcore_barrier()

        @pl.when(sid < 10)
        def _():
            pltpu.sync_copy(
                acc.at[pl.ds(sid * rpt, rpt)],
                out_hbm.at[cid].at[pl.ds(sid * rpt, rpt)],
            )

    return body(y, src2d, dst2d, zeros_nd)


def _dinv_of(degp_ref):
    deg = degp_ref[0] + degp_ref[1] + 1.0
    return lax.rsqrt(deg)


def _tc_xw(x, W1):
    def body(x_ref, w_ref, xw_ref):
        xw_ref[...] = jnp.dot(x_ref[...], w_ref[...], preferred_element_type=jnp.float32)

    return pl.pallas_call(
        body,
        out_shape=jax.ShapeDtypeStruct((N, D1), jnp.float32),
    )(x, W1)


def _tc_scale(xw, degp):
    def body(xw_ref, degp_ref, y1_ref):
        y1_ref[...] = xw_ref[...] * _dinv_of(degp_ref)

    return pl.pallas_call(
        body,
        out_shape=jax.ShapeDtypeStruct((N, D1), jnp.float32),
    )(xw, degp)


def _tc_mid(aggp, y1, degp, W2, b1):
    def body(aggp_ref, y1_ref, degp_ref, w_ref, b1_ref, y2_ref):
        dinv = _dinv_of(degp_ref)
        agg = aggp_ref[0] + aggp_ref[1] + y1_ref[...]
        pre = agg * dinv + b1_ref[...]
        h = jnp.where(pre > 0, pre, jnp.exp(jnp.minimum(pre, 0.0)) - 1.0)
        hw = jnp.dot(h, w_ref[...], preferred_element_type=jnp.float32)
        y2_ref[...] = hw * dinv[:, :1]

    return pl.pallas_call(
        body,
        out_shape=jax.ShapeDtypeStruct((N, D2P), jnp.float32),
    )(aggp, y1, degp, W2, b1)


def _tc_final(aggp2, y2, degp, b2p):
    def body(aggp_ref, y2_ref, degp_ref, b2_ref, out_ref):
        dinv = _dinv_of(degp_ref)
        agg = aggp_ref[0] + aggp_ref[1] + y2_ref[...]
        z = agg * dinv[:, :1] + b2_ref[...]
        lane = lax.broadcasted_iota(jnp.int32, (N, D2P), 1)
        zm = jnp.where(lane < 40, z, -1e30)
        m = jnp.max(zm, axis=1, keepdims=True)
        s = jnp.log(jnp.sum(jnp.exp(zm - m), axis=1, keepdims=True))
        out_ref[...] = (z - m - s)[:, :40]

    return pl.pallas_call(
        body,
        out_shape=jax.ShapeDtypeStruct((N, 40), jnp.float32),
    )(aggp2, y2, degp, b2p)


def kernel(node_feature, adj_mat, W1, b1, W2, b2):
    src2d = adj_mat[0].reshape(E // CH, CH)
    dst2d = adj_mat[1].reshape(E // CH, CH)
    ones_ch = jnp.ones((CH, D1), jnp.float32)
    zeros_n16 = jnp.zeros((N, D1), jnp.float32)
    zeros_n48 = jnp.zeros((N, D2P), jnp.float32)
    b1r = b1.reshape(1, D1)
    b2p = jnp.concatenate([b2, jnp.zeros((D2P - 40,), jnp.float32)]).reshape(1, D2P)

    xw = _tc_xw(node_feature, W1)
    degp = _sc_degree(dst2d, ones_ch, zeros_n16)
    y1 = _tc_scale(xw, degp)
    aggp1 = _sc_agg(y1, src2d, dst2d, zeros_n16, D1)
    y2 = _tc_mid(aggp1, y1, degp, W2, b1r)
    aggp2 = _sc_agg(y2, src2d, dst2d, zeros_n48, D2P)
    return _tc_final(aggp2, y2, degp, b2p)
